# baseline (speedup 1.0000x reference)
import functools

import jax
import jax.numpy as jnp
from jax import lax
from jax.experimental import pallas as pl
from jax.experimental.pallas import tpu as pltpu
from jax.experimental.pallas import tpu_sc as plsc

_VOCAB = 1000000
_EMB = 64
_BATCH = 16384
_KNEG = 20

_NC = 2
_NS = 16
_NW = _NC * _NS
_EPW = _BATCH // _NW
_CB = 16
_NCHUNK = _EPW // _CB
_RPC = _CB * _KNEG
_IDX_DMA = 80
_NDMA = _RPC // _IDX_DMA

_BT = 16384
_HB = _BT // 2
_NBT = -(-_VOCAB // _BT)
_ROWS = _NBT * _HB


def _relayout_call(inT, outT):
  def body(x_ref, y_ref, ox_ref, oy_ref):
    tx = jnp.transpose(x_ref[...])
    ty = jnp.transpose(y_ref[...])
    ox_ref[...] = jnp.concatenate([tx[:_HB], tx[_HB:]], axis=1)
    oy_ref[...] = jnp.concatenate([ty[:_HB], ty[_HB:]], axis=1)

  return pl.pallas_call(
      body,
      grid=(_NBT,),
      in_specs=[
          pl.BlockSpec((_EMB, _BT), lambda j: (0, j)),
          pl.BlockSpec((_EMB, _BT), lambda j: (0, j)),
      ],
      out_specs=[
          pl.BlockSpec((_HB, 128), lambda j: (j, 0)),
          pl.BlockSpec((_HB, 128), lambda j: (j, 0)),
      ],
      out_shape=[
          jax.ShapeDtypeStruct((_ROWS, 128), jnp.float32),
          jax.ShapeDtypeStruct((_ROWS, 128), jnp.float32),
      ],
  )(inT, outT)


def _scores_call(cen_p, cen_h, ctx_p, ctx_h, neg_p, neg_h, in128, out128):
  mesh = plsc.VectorSubcoreMesh(core_axis_name="c", subcore_axis_name="s")

  @functools.partial(
      pl.kernel,
      out_type=[
          jax.ShapeDtypeStruct((_BATCH,), jnp.float32),
          jax.ShapeDtypeStruct((_BATCH * _KNEG,), jnp.float32),
      ],
      mesh=mesh,
      scratch_types=[
          pltpu.VMEM((_EPW,), jnp.int32),
          pltpu.VMEM((_EPW,), jnp.int32),
          pltpu.VMEM((_EPW,), jnp.int32),
          pltpu.VMEM((_EPW,), jnp.int32),
          pltpu.VMEM((_EPW * _KNEG,), jnp.int32),
          pltpu.VMEM((_EPW * _KNEG,), jnp.int32),
          pltpu.VMEM((_CB, 128), jnp.float32),
          pltpu.VMEM((_CB, 128), jnp.float32),
          pltpu.VMEM((_RPC, 128), jnp.float32),
          pltpu.VMEM((_CB, 128), jnp.float32),
          pltpu.VMEM((_CB, 128), jnp.float32),
          pltpu.VMEM((_RPC, 128), jnp.float32),
          pltpu.VMEM((_CB,), jnp.float32),
          pltpu.VMEM((_RPC,), jnp.float32),
          pltpu.SemaphoreType.DMA,
          pltpu.SemaphoreType.DMA,
      ],
      compiler_params=pltpu.CompilerParams(
          needs_layout_passes=False, use_tc_tiling_on_sc=True),
  )
  def scores(cp_hbm, ch_hbm, op_hbm, oh_hbm, np_hbm, nh_hbm, in_hbm, out_hbm,
             pos_out, neg_out,
             idxc, hc_v, idxo, ho_v, idxn, hn_v,
             crowsA, orowsA, nrowsA, crowsB, orowsB, nrowsB,
             psc, nsc, semA, semB):
    wid = lax.axis_index("s") * _NC + lax.axis_index("c")
    base = wid * _EPW
    lanes = lax.iota(jnp.int32, 16)
    last_lane = lanes == 15
    gdn = lax.GatherDimensionNumbers(
        offset_dims=(), collapsed_slice_dims=(0,), start_index_map=(0,))
    perms = [(lanes ^ sh)[:, None] for sh in (8, 4, 2, 1)]

    def lane_sum(v):
      for p in perms:
        v = v + lax.gather(v, p, gdn, slice_sizes=(1,),
                           mode=lax.GatherScatterMode.PROMISE_IN_BOUNDS)
      return v
    pltpu.sync_copy(cp_hbm.at[pl.ds(base, _EPW)], idxc)
    pltpu.sync_copy(ch_hbm.at[pl.ds(base, _EPW)], hc_v)
    pltpu.sync_copy(op_hbm.at[pl.ds(base, _EPW)], idxo)
    pltpu.sync_copy(oh_hbm.at[pl.ds(base, _EPW)], ho_v)
    pltpu.sync_copy(np_hbm.at[pl.ds(base * _KNEG, _EPW * _KNEG)], idxn)
    pltpu.sync_copy(nh_hbm.at[pl.ds(base * _KNEG, _EPW * _KNEG)], hn_v)

    def issue(c, cr, orr, nr, sem):
      off = c * _CB
      pltpu.async_copy(in_hbm.at[idxc.at[pl.ds(off, _CB)]], cr, sem)
      pltpu.async_copy(out_hbm.at[idxo.at[pl.ds(off, _CB)]], orr, sem)
      for j in range(_NDMA):
        pltpu.async_copy(
            out_hbm.at[idxn.at[pl.ds(c * _RPC + j * _IDX_DMA, _IDX_DMA)]],
            nr.at[pl.ds(j * _IDX_DMA, _IDX_DMA)], sem)

    def drain(cr, orr, nr, sem):
      pltpu.make_async_copy(in_hbm.at[idxc.at[pl.ds(0, _CB)]], cr, sem).wait()
      pltpu.make_async_copy(out_hbm.at[idxo.at[pl.ds(0, _CB)]], orr, sem).wait()
      for j in range(_NDMA):
        pltpu.make_async_copy(
            out_hbm.at[idxn.at[pl.ds(j * _IDX_DMA, _IDX_DMA)]],
            nr.at[pl.ds(j * _IDX_DMA, _IDX_DMA)], sem).wait()

    def compute(c, cr, orr, nr):
      off = c * _CB

      def b_body(b, carry2):
        hcb = hc_v[pl.ds(off + b, 16)][0]
        hob = ho_v[pl.ds(off + b, 16)][0]
        hn1 = hn_v[pl.ds((off + b) * _KNEG, 16)]
        hn2 = hn_v[pl.ds((off + b) * _KNEG + 4, 16)]
        vc = [cr[b, pl.ds(hcb + j * 16, 16)] for j in range(4)]
        vo = [orr[b, pl.ds(hob + j * 16, 16)] for j in range(4)]
        s = vc[0] * vo[0] + vc[1] * vo[1] + vc[2] * vo[2] + vc[3] * vo[3]
        plsc.store_scatter(psc, [jnp.full((16,), b, jnp.int32)],
                           lane_sum(s), mask=last_lane)
        for kk in range(_KNEG):
          r = b * _KNEG + kk
          hnb = hn1[kk] if kk < 16 else hn2[kk - 4]
          nv = [nr[r, pl.ds(hnb + j * 16, 16)] for j in range(4)]
          t = vc[0] * nv[0] + vc[1] * nv[1] + vc[2] * nv[2] + vc[3] * nv[3]
          plsc.store_scatter(nsc, [jnp.full((16,), r, jnp.int32)],
                             lane_sum(t), mask=last_lane)
        return carry2

      lax.fori_loop(0, _CB, b_body, 0)
      pltpu.sync_copy(psc, pos_out.at[pl.ds(base + off, _CB)])
      pltpu.sync_copy(nsc, neg_out.at[pl.ds((base + off) * _KNEG, _RPC)])

    issue(0, crowsA, orowsA, nrowsA, semA)

    def pair_body(c2, carry):
      c = c2 * 2
      issue(c + 1, crowsB, orowsB, nrowsB, semB)
      drain(crowsA, orowsA, nrowsA, semA)
      compute(c, crowsA, orowsA, nrowsA)
      issue(jnp.minimum(c + 2, _NCHUNK - 1), crowsA, orowsA, nrowsA, semA)
      drain(crowsB, orowsB, nrowsB, semB)
      compute(c + 1, crowsB, orowsB, nrowsB)
      return carry

    lax.fori_loop(0, _NCHUNK // 2, pair_body, 0)
    drain(crowsA, orowsA, nrowsA, semA)

  return scores(cen_p, cen_h, ctx_p, ctx_h, neg_p, neg_h, in128, out128)


def _loss_call(pos2d, neg2d):
  def body(pos_ref, neg_ref, out_ref):
    p = pos_ref[...]
    n = neg_ref[...]

    def logsig(x):
      return jnp.minimum(x, 0.0) - jnp.log1p(jnp.exp(-jnp.abs(x)))

    tot = jnp.sum(logsig(p)) + jnp.sum(logsig(-n))
    out_ref[...] = jnp.full((1, 1), -tot / _BATCH, jnp.float32)

  return pl.pallas_call(
      body,
      out_shape=jax.ShapeDtypeStruct((1, 1), jnp.float32),
  )(pos2d, neg2d)


def _packed(idx):
  p = (idx // _BT) * _HB + (idx % _HB)
  h = ((idx // _HB) % 2) * 64
  return p, h


def kernel(center, context, negatives, in_embed, out_embed):
  cen_p, cen_h = _packed(center.astype(jnp.int32))
  ctx_p, ctx_h = _packed(context.astype(jnp.int32))
  neg_p, neg_h = _packed(negatives.astype(jnp.int32).reshape(-1))
  in128, out128 = _relayout_call(in_embed.T, out_embed.T)
  pos, negs = _scores_call(cen_p, cen_h, ctx_p, ctx_h, neg_p, neg_h,
                           in128, out128)
  loss = _loss_call(pos.reshape(128, 128), negs.reshape(2560, 128))
  return loss[0, 0]


# speedup vs baseline: 1.0281x; 1.0281x over previous
import functools

import jax
import jax.numpy as jnp
from jax import lax
from jax.experimental import pallas as pl
from jax.experimental.pallas import tpu as pltpu
from jax.experimental.pallas import tpu_sc as plsc

_VOCAB = 1000000
_EMB = 64
_BATCH = 16384
_KNEG = 20

_NC = 2
_NS = 16
_NW = _NC * _NS
_EPW = _BATCH // _NW
_CB = 16
_NCHUNK = _EPW // _CB
_RPC = _CB * _KNEG
_IDX_DMA = 80
_NDMA = _RPC // _IDX_DMA

_BT = 16384
_HB = _BT // 2
_NBT = -(-_VOCAB // _BT)
_ROWS = _NBT * _HB


def _relayout_call(inT, outT):
  def body(x_ref, y_ref, ox_ref, oy_ref):
    tx = jnp.transpose(x_ref[...])
    ty = jnp.transpose(y_ref[...])
    ox_ref[...] = jnp.concatenate([tx[:_HB], tx[_HB:]], axis=1)
    oy_ref[...] = jnp.concatenate([ty[:_HB], ty[_HB:]], axis=1)

  return pl.pallas_call(
      body,
      grid=(_NBT,),
      in_specs=[
          pl.BlockSpec((_EMB, _BT), lambda j: (0, j)),
          pl.BlockSpec((_EMB, _BT), lambda j: (0, j)),
      ],
      out_specs=[
          pl.BlockSpec((_HB, 128), lambda j: (j, 0)),
          pl.BlockSpec((_HB, 128), lambda j: (j, 0)),
      ],
      out_shape=[
          jax.ShapeDtypeStruct((_ROWS, 128), jnp.float32),
          jax.ShapeDtypeStruct((_ROWS, 128), jnp.float32),
      ],
  )(inT, outT)


def _scores_call(cen_p, cen_h, ctx_p, ctx_h, neg_p, neg_h, in128, out128):
  mesh = plsc.VectorSubcoreMesh(core_axis_name="c", subcore_axis_name="s")

  @functools.partial(
      pl.kernel,
      out_type=[
          jax.ShapeDtypeStruct((_BATCH,), jnp.float32),
          jax.ShapeDtypeStruct((_BATCH * _KNEG,), jnp.float32),
      ],
      mesh=mesh,
      scratch_types=[
          pltpu.VMEM((_EPW,), jnp.int32),
          pltpu.VMEM((_EPW,), jnp.int32),
          pltpu.VMEM((_EPW,), jnp.int32),
          pltpu.VMEM((_EPW,), jnp.int32),
          pltpu.VMEM((_EPW * _KNEG,), jnp.int32),
          pltpu.VMEM((_EPW * _KNEG,), jnp.int32),
          pltpu.VMEM((_CB, 128), jnp.float32),
          pltpu.VMEM((_CB, 128), jnp.float32),
          pltpu.VMEM((_RPC, 128), jnp.float32),
          pltpu.VMEM((_CB, 128), jnp.float32),
          pltpu.VMEM((_CB, 128), jnp.float32),
          pltpu.VMEM((_RPC, 128), jnp.float32),
          pltpu.VMEM((_CB,), jnp.float32),
          pltpu.VMEM((_RPC,), jnp.float32),
          pltpu.SemaphoreType.DMA,
          pltpu.SemaphoreType.DMA,
      ],
      compiler_params=pltpu.CompilerParams(
          needs_layout_passes=False, use_tc_tiling_on_sc=True),
  )
  def scores(cp_hbm, ch_hbm, op_hbm, oh_hbm, np_hbm, nh_hbm, in_hbm, out_hbm,
             pos_out, neg_out,
             idxc, hc_v, idxo, ho_v, idxn, hn_v,
             crowsA, orowsA, nrowsA, crowsB, orowsB, nrowsB,
             psc, nsc, semA, semB):
    wid = lax.axis_index("s") * _NC + lax.axis_index("c")
    base = wid * _EPW
    lanes = lax.iota(jnp.int32, 16)
    last_lane = lanes == 15
    pltpu.sync_copy(cp_hbm.at[pl.ds(base, _EPW)], idxc)
    pltpu.sync_copy(ch_hbm.at[pl.ds(base, _EPW)], hc_v)
    pltpu.sync_copy(op_hbm.at[pl.ds(base, _EPW)], idxo)
    pltpu.sync_copy(oh_hbm.at[pl.ds(base, _EPW)], ho_v)
    pltpu.sync_copy(np_hbm.at[pl.ds(base * _KNEG, _EPW * _KNEG)], idxn)
    pltpu.sync_copy(nh_hbm.at[pl.ds(base * _KNEG, _EPW * _KNEG)], hn_v)

    def issue(c, cr, orr, nr, sem):
      off = c * _CB
      pltpu.async_copy(in_hbm.at[idxc.at[pl.ds(off, _CB)]], cr, sem)
      pltpu.async_copy(out_hbm.at[idxo.at[pl.ds(off, _CB)]], orr, sem)
      for j in range(_NDMA):
        pltpu.async_copy(
            out_hbm.at[idxn.at[pl.ds(c * _RPC + j * _IDX_DMA, _IDX_DMA)]],
            nr.at[pl.ds(j * _IDX_DMA, _IDX_DMA)], sem)

    def drain(cr, orr, nr, sem):
      pltpu.make_async_copy(in_hbm.at[idxc.at[pl.ds(0, _CB)]], cr, sem).wait()
      pltpu.make_async_copy(out_hbm.at[idxo.at[pl.ds(0, _CB)]], orr, sem).wait()
      for j in range(_NDMA):
        pltpu.make_async_copy(
            out_hbm.at[idxn.at[pl.ds(j * _IDX_DMA, _IDX_DMA)]],
            nr.at[pl.ds(j * _IDX_DMA, _IDX_DMA)], sem).wait()

    def compute(c, cr, orr, nr):
      off = c * _CB

      def b_body(b, carry2):
        hcb = hc_v[pl.ds(off + b, 16)][0]
        hob = ho_v[pl.ds(off + b, 16)][0]
        hn1 = hn_v[pl.ds((off + b) * _KNEG, 16)]
        hn2 = hn_v[pl.ds((off + b) * _KNEG + 4, 16)]
        vc = [cr[b, pl.ds(hcb + j * 16, 16)] for j in range(4)]
        vo = [orr[b, pl.ds(hob + j * 16, 16)] for j in range(4)]
        s = vc[0] * vo[0] + vc[1] * vo[1] + vc[2] * vo[2] + vc[3] * vo[3]
        plsc.store_scatter(psc, [jnp.full((16,), b, jnp.int32)],
                           jnp.full((16,), jnp.sum(s)), mask=last_lane)
        for kk in range(_KNEG):
          r = b * _KNEG + kk
          hnb = hn1[kk] if kk < 16 else hn2[kk - 4]
          nv = [nr[r, pl.ds(hnb + j * 16, 16)] for j in range(4)]
          t = vc[0] * nv[0] + vc[1] * nv[1] + vc[2] * nv[2] + vc[3] * nv[3]
          plsc.store_scatter(nsc, [jnp.full((16,), r, jnp.int32)],
                             jnp.full((16,), jnp.sum(t)), mask=last_lane)
        return carry2

      lax.fori_loop(0, _CB, b_body, 0)
      pltpu.sync_copy(psc, pos_out.at[pl.ds(base + off, _CB)])
      pltpu.sync_copy(nsc, neg_out.at[pl.ds((base + off) * _KNEG, _RPC)])

    issue(0, crowsA, orowsA, nrowsA, semA)

    def pair_body(c2, carry):
      c = c2 * 2
      issue(c + 1, crowsB, orowsB, nrowsB, semB)
      drain(crowsA, orowsA, nrowsA, semA)
      compute(c, crowsA, orowsA, nrowsA)
      issue(jnp.minimum(c + 2, _NCHUNK - 1), crowsA, orowsA, nrowsA, semA)
      drain(crowsB, orowsB, nrowsB, semB)
      compute(c + 1, crowsB, orowsB, nrowsB)
      return carry

    lax.fori_loop(0, _NCHUNK // 2, pair_body, 0)
    drain(crowsA, orowsA, nrowsA, semA)

  return scores(cen_p, cen_h, ctx_p, ctx_h, neg_p, neg_h, in128, out128)


def _loss_call(pos2d, neg2d):
  def body(pos_ref, neg_ref, out_ref):
    p = pos_ref[...]
    n = neg_ref[...]

    def logsig(x):
      return jnp.minimum(x, 0.0) - jnp.log1p(jnp.exp(-jnp.abs(x)))

    tot = jnp.sum(logsig(p)) + jnp.sum(logsig(-n))
    out_ref[...] = jnp.full((1, 1), -tot / _BATCH, jnp.float32)

  return pl.pallas_call(
      body,
      out_shape=jax.ShapeDtypeStruct((1, 1), jnp.float32),
  )(pos2d, neg2d)


def _packed(idx):
  p = (idx // _BT) * _HB + (idx % _HB)
  h = ((idx // _HB) % 2) * 64
  return p, h


def kernel(center, context, negatives, in_embed, out_embed):
  cen_p, cen_h = _packed(center.astype(jnp.int32))
  ctx_p, ctx_h = _packed(context.astype(jnp.int32))
  neg_p, neg_h = _packed(negatives.astype(jnp.int32).reshape(-1))
  in128, out128 = _relayout_call(in_embed.T, out_embed.T)
  pos, negs = _scores_call(cen_p, cen_h, ctx_p, ctx_h, neg_p, neg_h,
                           in128, out128)
  loss = _loss_call(pos.reshape(128, 128), negs.reshape(2560, 128))
  return loss[0, 0]
